# Initial kernel scaffold; baseline (speedup 1.0000x reference)
import jax
import jax.numpy as jnp
from jax.experimental import pallas as pl


def kernel(x, a, W1, b1, gamma, beta, Wp1, bp1, Wp2, bp2):
    raise NotImplementedError("write your pallas kernel here")


# speedup vs baseline: 1.1076x; 1.1076x over previous
import jax
import jax.numpy as jnp
from jax.experimental import pallas as pl
from jax.experimental.pallas import tpu as pltpu

_N = 10000
_BR = 200
_NB = _N // _BR


def _fused_kernel(x_ref, a_ref, W1_ref, b1_ref, gamma_ref, beta_ref,
                  Wp1_ref, bp1_ref, Wp2_ref, bp2_ref,
                  zn_ref, p_ref,
                  z0_s, z1_s, z2_s):
    ph = pl.program_id(0)
    r = pl.program_id(1)

    @pl.when((ph == 0) & (r == 0))
    def _init():
        z0_s[...] = jnp.maximum(
            jnp.dot(x_ref[...], W1_ref[...], preferred_element_type=jnp.float32)
            + b1_ref[...], 0.0)

    @pl.when(ph == 0)
    def _pass1():
        z1_s[pl.ds(r * _BR, _BR), :] = jnp.dot(
            a_ref[...], z0_s[...], preferred_element_type=jnp.float32)

    @pl.when(ph == 1)
    def _pass2():
        z2_s[pl.ds(r * _BR, _BR), :] = jnp.dot(
            a_ref[...], z1_s[...], preferred_element_type=jnp.float32)

    @pl.when((ph == 1) & (r == _NB - 1))
    def _finish():
        z2 = z2_s[...]
        mean = jnp.mean(z2, axis=0, keepdims=True)
        var = jnp.mean((z2 - mean) ** 2, axis=0, keepdims=True)
        zn = (z2 - mean) * jax.lax.rsqrt(var + 1e-5) * gamma_ref[...] + beta_ref[...]
        zn_ref[...] = zn
        h = jnp.maximum(
            jnp.dot(zn, Wp1_ref[...], preferred_element_type=jnp.float32)
            + bp1_ref[...], 0.0)
        p_ref[...] = jnp.dot(
            h, Wp2_ref[...], preferred_element_type=jnp.float32) + bp2_ref[...]


def kernel(x, a, W1, b1, gamma, beta, Wp1, bp1, Wp2, bp2):
    emb = W1.shape[1]
    proj = Wp1.shape[1]

    def const(shape):
        return pl.BlockSpec(shape, lambda p, r: (0, 0))

    zn, p = pl.pallas_call(
        _fused_kernel,
        grid=(2, _NB),
        in_specs=[
            const(x.shape),
            pl.BlockSpec((_BR, _N), lambda p, r: (r, 0)),
            const(W1.shape), const((1, emb)), const((1, emb)), const((1, emb)),
            const(Wp1.shape), const((1, proj)), const(Wp2.shape),
            const((1, proj)),
        ],
        out_specs=[const((_N, emb)), const((_N, proj))],
        out_shape=[jax.ShapeDtypeStruct((_N, emb), jnp.float32),
                   jax.ShapeDtypeStruct((_N, proj), jnp.float32)],
        scratch_shapes=[pltpu.VMEM((_N, emb), jnp.float32),
                        pltpu.VMEM((_N, emb), jnp.float32),
                        pltpu.VMEM((_N, emb), jnp.float32)],
    )(x, a, W1, b1.reshape(1, -1), gamma.reshape(1, -1), beta.reshape(1, -1),
      Wp1, bp1.reshape(1, -1), Wp2, bp2.reshape(1, -1))
    return (zn, p)
